# baseline (speedup 1.0000x reference)
import functools

import jax
import jax.numpy as jnp
from jax import lax
from jax.experimental import pallas as pl
from jax.experimental.pallas import tpu as pltpu
from jax.experimental.pallas import tpu_sc as plsc

N = 10000
E = 320000
NC = 2
NS = 16
NW = NC * NS
EP = E // NW
CH = 128
NCH = 79
PADE = NCH * CH - EP
NPAD = 10240
ROWS_PER_TILE = NPAD // NS
AD = 64

_mesh = plsc.VectorSubcoreMesh(core_axis_name="c", subcore_axis_name="s")
_sc_params = pltpu.CompilerParams(use_tc_tiling_on_sc=False)


@functools.partial(
    pl.kernel,
    out_type=jax.ShapeDtypeStruct((NC * NPAD,), jnp.float32),
    mesh=_mesh,
    compiler_params=_sc_params,
    scratch_types=[
        pltpu.VMEM((NCH, CH), jnp.int32),
        pltpu.VMEM((CH,), jnp.float32),
        pltpu.VMEM((ROWS_PER_TILE,), jnp.float32),
        pltpu.VMEM_SHARED((NPAD,), jnp.float32),
    ],
)
def _deg_kernel(dst_hbm, out_hbm, didx, ones_v, zbuf, deg_sp):
    c = lax.axis_index("c")
    s = lax.axis_index("s")
    wid = c * NS + s

    def zstore(i, _):
        zbuf[pl.ds(i * 16, 16)] = jnp.zeros((16,), jnp.float32)
        return 0

    lax.fori_loop(0, ROWS_PER_TILE // 16, zstore, 0)

    def fill_ones(i, _):
        ones_v[pl.ds(i * 16, 16)] = jnp.ones((16,), jnp.float32)
        return 0

    lax.fori_loop(0, CH // 16, fill_ones, 0)

    pltpu.sync_copy(zbuf, deg_sp.at[pl.ds(s * ROWS_PER_TILE, ROWS_PER_TILE)])
    plsc.subcore_barrier()

    pltpu.sync_copy(dst_hbm.at[wid], didx)

    def body(j, _):
        pltpu.sync_copy(ones_v, deg_sp.at[didx.at[j]], add=True)
        return 0

    lax.fori_loop(0, NCH, body, 0)

    plsc.subcore_barrier()

    pltpu.sync_copy(deg_sp.at[pl.ds(s * ROWS_PER_TILE, ROWS_PER_TILE)], zbuf)
    pltpu.sync_copy(
        zbuf, out_hbm.at[pl.ds(c * NPAD + s * ROWS_PER_TILE, ROWS_PER_TILE)])


def _make_agg(P):
    zr = 128

    @functools.partial(
        pl.kernel,
        out_type=jax.ShapeDtypeStruct((NC, P, NPAD, AD), jnp.float32),
        mesh=_mesh,
        compiler_params=_sc_params,
        scratch_types=[
            pltpu.VMEM((NCH, CH), jnp.int32),
            pltpu.VMEM((NCH, CH), jnp.int32),
            pltpu.VMEM((CH, AD), jnp.float32),
            pltpu.VMEM((CH, AD), jnp.float32),
            pltpu.VMEM((zr, AD), jnp.float32),
            pltpu.VMEM_SHARED((NPAD, AD), jnp.float32),
            pltpu.SemaphoreType.DMA,
            pltpu.SemaphoreType.DMA,
        ],
    )
    def agg(*args):
        ys = args[:P]
        (src_hbm, dst_hbm, out_hbm,
         sidx, didx, rows0, rows1, zbuf, acc, sem0, sem1) = args[P:]
        c = lax.axis_index("c")
        s = lax.axis_index("s")
        wid = c * NS + s

        cols = AD // 16

        def zstore(t, _):
            zbuf[t // cols, pl.ds((t % cols) * 16, 16)] = jnp.zeros(
                (16,), jnp.float32)
            return 0

        lax.fori_loop(0, zr * cols, zstore, 0)

        pltpu.sync_copy(src_hbm.at[wid], sidx)
        pltpu.sync_copy(dst_hbm.at[wid], didx)

        for p in range(P):
            for k in range(ROWS_PER_TILE // zr):
                pltpu.sync_copy(
                    zbuf, acc.at[pl.ds(s * ROWS_PER_TILE + k * zr, zr)])
            plsc.subcore_barrier()

            y_hbm = ys[p]

            def gather_start(j, buf, sem):
                return pltpu.async_copy(y_hbm.at[sidx.at[j]], buf, sem)

            def gather_wait(j, buf, sem):
                pltpu.make_async_copy(y_hbm.at[sidx.at[j]], buf, sem).wait()

            def scat(j, buf):
                pltpu.sync_copy(buf, acc.at[didx.at[j]], add=True)

            gather_start(0, rows0, sem0)

            def body(k, _):
                j0 = 2 * k
                gather_start(j0 + 1, rows1, sem1)
                gather_wait(j0, rows0, sem0)
                scat(j0, rows0)
                gather_start(j0 + 2, rows0, sem0)
                gather_wait(j0 + 1, rows1, sem1)
                scat(j0 + 1, rows1)
                return 0

            lax.fori_loop(0, (NCH - 1) // 2, body, 0)

            gather_wait(NCH - 1, rows0, sem0)
            scat(NCH - 1, rows0)

            plsc.subcore_barrier()

            for k in range(ROWS_PER_TILE // zr):
                base = s * ROWS_PER_TILE + k * zr
                pltpu.sync_copy(acc.at[pl.ds(base, zr)], zbuf)
                pltpu.sync_copy(zbuf, out_hbm.at[c, p, pl.ds(base, zr)])
            if p + 1 < P:
                lax.fori_loop(0, zr * cols, zstore, 0)

    return agg


_aggA = _make_agg(1)
_aggB = _make_agg(1)


def _tc1a_body(x_ref, w_ref, d0_ref, d1_ref, ya_ref, dinv_ref):
    deg = d0_ref[0:N, :] + d1_ref[0:N, :] + 1.0
    dinv = lax.rsqrt(deg)
    dinv_ref[...] = dinv
    xw = jnp.dot(x_ref[...], w_ref[...],
                 preferred_element_type=jnp.float32,
                 precision=lax.Precision.HIGHEST)
    ya_ref[...] = xw * dinv


def _tc1b_body(x_ref, w_ref, dinv_ref, yb_ref):
    xw = jnp.dot(x_ref[...], w_ref[...],
                 preferred_element_type=jnp.float32,
                 precision=lax.Precision.HIGHEST)
    yb_ref[...] = xw * dinv_ref[...]


_SELU_SCALE = 1.0507009873554804934193349852946
_SELU_ALPHA = 1.6732632423543772848170429916717


def _selu(pre):
    return _SELU_SCALE * jnp.where(
        pre > 0.0, pre, _SELU_ALPHA * (jnp.exp(pre) - 1.0))


def _tc2a_body(q0_ref, q1_ref, ya_ref, dinv_ref, b_ref, w_ref, p_ref):
    pre = dinv_ref[...] * (q0_ref[...] + q1_ref[...] + ya_ref[...]) + b_ref[...]
    p_ref[...] = jnp.dot(_selu(pre), w_ref[...],
                         preferred_element_type=jnp.float32,
                         precision=lax.Precision.HIGHEST)


def _tc2b_body(q0_ref, q1_ref, yb_ref, dinv_ref, b_ref, w_ref, p_ref,
               y2_ref):
    pre = dinv_ref[...] * (q0_ref[...] + q1_ref[...] + yb_ref[...]) + b_ref[...]
    hw = jnp.dot(_selu(pre), w_ref[...],
                 preferred_element_type=jnp.float32,
                 precision=lax.Precision.HIGHEST)
    y2_ref[...] = (p_ref[...] + hw) * dinv_ref[...]


def _tc3_body(q0_ref, q1_ref, y2_ref, dinv_ref, b2_ref, out_ref):
    o = (dinv_ref[...] * (q0_ref[...] + q1_ref[...] + y2_ref[...])
         + b2_ref[...])
    m = jnp.max(o, axis=1, keepdims=True)
    lse = m + jnp.log(jnp.sum(jnp.exp(o - m), axis=1, keepdims=True))
    out_ref[...] = o - lse


def kernel(x, edge_index, W1, b1, W2, b2):
    D_h = W1.shape[1]
    D_out = W2.shape[1]

    i = jnp.arange(PADE, dtype=jnp.int32)[None, :]
    w = jnp.arange(NW, dtype=jnp.int32)[:, None]
    pad_src = (i * 83 + w * 41) % N
    pad_dst = N + (i + w * 7) % (NPAD - N)
    src3 = jnp.concatenate(
        [edge_index[0].reshape(NW, EP), pad_src], axis=1).reshape(NW, NCH, CH)
    dst3 = jnp.concatenate(
        [edge_index[1].reshape(NW, EP), pad_dst], axis=1).reshape(NW, NCH, CH)

    degp = _deg_kernel(dst3)
    d0 = degp[:NPAD].reshape(NPAD, 1)
    d1 = degp[NPAD:].reshape(NPAD, 1)

    ya, dinv = pl.pallas_call(
        _tc1a_body,
        out_shape=[
            jax.ShapeDtypeStruct((N, AD), jnp.float32),
            jax.ShapeDtypeStruct((N, 1), jnp.float32),
        ],
    )(x, W1[:, :AD], d0, d1)

    r1a = _aggA(ya, src3, dst3)

    yb = pl.pallas_call(
        _tc1b_body,
        out_shape=jax.ShapeDtypeStruct((N, AD), jnp.float32),
    )(x, W1[:, AD:], dinv)

    r1b = _aggB(yb, src3, dst3)

    R = 2000
    _rows64 = pl.BlockSpec((R, AD), lambda i: (i, 0))
    _rows1 = pl.BlockSpec((R, 1), lambda i: (i, 0))
    _bcast = lambda shape: pl.BlockSpec(shape, lambda i: (0, 0))

    pa = pl.pallas_call(
        _tc2a_body,
        grid=(N // R,),
        in_specs=[_rows64, _rows64, _rows64, _rows1,
                  _bcast((1, AD)), _bcast((AD, D_out))],
        out_specs=pl.BlockSpec((R, D_out), lambda i: (i, 0)),
        out_shape=jax.ShapeDtypeStruct((N, D_out), jnp.float32),
    )(r1a[0, 0], r1a[1, 0], ya, dinv,
      b1[:AD].reshape(1, AD), W2[:AD])

    y2 = pl.pallas_call(
        _tc2b_body,
        grid=(N // R,),
        in_specs=[_rows64, _rows64, _rows64, _rows1,
                  _bcast((1, AD)), _bcast((AD, D_out)),
                  pl.BlockSpec((R, D_out), lambda i: (i, 0))],
        out_specs=pl.BlockSpec((R, D_out), lambda i: (i, 0)),
        out_shape=jax.ShapeDtypeStruct((N, D_out), jnp.float32),
    )(r1b[0, 0], r1b[1, 0], yb, dinv,
      b1[AD:].reshape(1, AD), W2[AD:], pa)

    r2 = _aggA(y2, src3, dst3)

    out = pl.pallas_call(
        _tc3_body,
        grid=(N // R,),
        in_specs=[_rows64, _rows64, _rows64, _rows1,
                  pl.BlockSpec((1, D_out), lambda i: (0, 0))],
        out_specs=pl.BlockSpec((R, D_out), lambda i: (i, 0)),
        out_shape=jax.ShapeDtypeStruct((N, D_out), jnp.float32),
    )(r2[0, 0], r2[1, 0], y2, dinv, b2.reshape(1, D_out))

    return out


# speedup vs baseline: 1.0736x; 1.0736x over previous
import functools

import jax
import jax.numpy as jnp
from jax import lax
from jax.experimental import pallas as pl
from jax.experimental.pallas import tpu as pltpu
from jax.experimental.pallas import tpu_sc as plsc

N = 10000
E = 320000
NC = 2
NS = 16
NW = NC * NS
EP = E // NW
CH = 80
NCH = 125
NPAD = 10240
ROWS_PER_TILE = NPAD // NS

_mesh = plsc.VectorSubcoreMesh(core_axis_name="c", subcore_axis_name="s")
_sc_params = pltpu.CompilerParams(use_tc_tiling_on_sc=False)


@functools.partial(
    pl.kernel,
    out_type=jax.ShapeDtypeStruct((NC * NPAD,), jnp.float32),
    mesh=_mesh,
    compiler_params=_sc_params,
    scratch_types=[
        pltpu.VMEM((NCH, CH), jnp.int32),
        pltpu.VMEM((CH,), jnp.float32),
        pltpu.VMEM((ROWS_PER_TILE,), jnp.float32),
        pltpu.VMEM_SHARED((NPAD,), jnp.float32),
    ],
)
def _deg_kernel(dst_hbm, out_hbm, didx, ones_v, zbuf, deg_sp):
    c = lax.axis_index("c")
    s = lax.axis_index("s")
    wid = c * NS + s

    def zstore(i, _):
        zbuf[pl.ds(i * 16, 16)] = jnp.zeros((16,), jnp.float32)
        return 0

    lax.fori_loop(0, ROWS_PER_TILE // 16, zstore, 0)

    def fill_ones(i, _):
        ones_v[pl.ds(i * 16, 16)] = jnp.ones((16,), jnp.float32)
        return 0

    lax.fori_loop(0, CH // 16, fill_ones, 0)

    pltpu.sync_copy(zbuf, deg_sp.at[pl.ds(s * ROWS_PER_TILE, ROWS_PER_TILE)])
    plsc.subcore_barrier()

    pltpu.sync_copy(dst_hbm.at[wid], didx)

    def body(j, _):
        pltpu.sync_copy(ones_v, deg_sp.at[didx.at[j]], add=True)
        return 0

    lax.fori_loop(0, NCH, body, 0)

    plsc.subcore_barrier()

    pltpu.sync_copy(deg_sp.at[pl.ds(s * ROWS_PER_TILE, ROWS_PER_TILE)], zbuf)
    pltpu.sync_copy(
        zbuf, out_hbm.at[pl.ds(c * NPAD + s * ROWS_PER_TILE, ROWS_PER_TILE)])


def _make_agg(W):
    zr = 32
    cols = W // 16

    @functools.partial(
        pl.kernel,
        out_type=jax.ShapeDtypeStruct((NC, NPAD, W), jnp.float32),
        mesh=_mesh,
        compiler_params=_sc_params,
        scratch_types=[
            pltpu.VMEM((NCH, CH), jnp.int32),
            pltpu.VMEM((NCH, CH), jnp.int32),
            pltpu.VMEM((CH, W), jnp.float32),
            pltpu.VMEM((CH, W), jnp.float32),
            pltpu.VMEM((zr, W), jnp.float32),
            pltpu.VMEM_SHARED((NPAD, W), jnp.float32),
            pltpu.SemaphoreType.DMA,
            pltpu.SemaphoreType.DMA,
        ],
    )
    def agg(y_hbm, src_hbm, dst_hbm, out_hbm,
            sidx, didx, rows0, rows1, zbuf, acc, sem0, sem1):
        c = lax.axis_index("c")
        s = lax.axis_index("s")
        wid = c * NS + s

        def zstore(t, _):
            zbuf[t // cols, pl.ds((t % cols) * 16, 16)] = jnp.zeros(
                (16,), jnp.float32)
            return 0

        lax.fori_loop(0, zr * cols, zstore, 0)

        pltpu.sync_copy(src_hbm.at[wid], sidx)
        pltpu.sync_copy(dst_hbm.at[wid], didx)

        for k in range(ROWS_PER_TILE // zr):
            pltpu.sync_copy(
                zbuf, acc.at[pl.ds(s * ROWS_PER_TILE + k * zr, zr)])
        plsc.subcore_barrier()

        def gather_start(j, buf, sem):
            return pltpu.async_copy(y_hbm.at[sidx.at[j]], buf, sem)

        def gather_wait(j, buf, sem):
            pltpu.make_async_copy(y_hbm.at[sidx.at[j]], buf, sem).wait()

        def scat(j, buf):
            pltpu.sync_copy(buf, acc.at[didx.at[j]], add=True)

        gather_start(0, rows0, sem0)

        def body(k, _):
            j0 = 2 * k
            gather_start(j0 + 1, rows1, sem1)
            gather_wait(j0, rows0, sem0)
            scat(j0, rows0)
            gather_start(j0 + 2, rows0, sem0)
            gather_wait(j0 + 1, rows1, sem1)
            scat(j0 + 1, rows1)
            return 0

        lax.fori_loop(0, (NCH - 1) // 2, body, 0)

        gather_wait(NCH - 1, rows0, sem0)
        scat(NCH - 1, rows0)

        plsc.subcore_barrier()

        for k in range(ROWS_PER_TILE // zr):
            base = s * ROWS_PER_TILE + k * zr
            pltpu.sync_copy(acc.at[pl.ds(base, zr)], zbuf)
            pltpu.sync_copy(zbuf, out_hbm.at[c, pl.ds(base, zr)])

    return agg


_agg = _make_agg(128)


def _tc1_body(x_ref, w_ref, d0_ref, d1_ref, y_ref, dinv_ref):
    deg = d0_ref[0:N, :] + d1_ref[0:N, :] + 1.0
    dinv = lax.rsqrt(deg)
    dinv_ref[...] = dinv
    xw = jnp.dot(x_ref[...], w_ref[...],
                 preferred_element_type=jnp.float32,
                 precision=lax.Precision.HIGHEST)
    y_ref[...] = xw * dinv


_SELU_SCALE = 1.0507009873554804934193349852946
_SELU_ALPHA = 1.6732632423543772848170429916717


def _tc2_body(q0_ref, q1_ref, y_ref, dinv_ref, b1_ref, w2_ref, y2_ref):
    dinv = dinv_ref[...]
    agg = q0_ref[0] + q1_ref[0] + y_ref[...]
    pre = dinv * agg + b1_ref[...]
    h = _SELU_SCALE * jnp.where(
        pre > 0.0, pre, _SELU_ALPHA * (jnp.exp(pre) - 1.0))
    hw = jnp.dot(h, w2_ref[...],
                 preferred_element_type=jnp.float32,
                 precision=lax.Precision.HIGHEST)
    y2_ref[...] = jnp.concatenate(
        [hw * dinv, jnp.zeros_like(hw)], axis=1)


def _tc3_body(q0_ref, q1_ref, y2_ref, dinv_ref, b2_ref, out_ref):
    d = out_ref.shape[1]
    o = (dinv_ref[...] * (q0_ref[0, :, :d] + q1_ref[0, :, :d]
                          + y2_ref[:, :d]) + b2_ref[...])
    m = jnp.max(o, axis=1, keepdims=True)
    lse = m + jnp.log(jnp.sum(jnp.exp(o - m), axis=1, keepdims=True))
    out_ref[...] = o - lse


def kernel(x, edge_index, W1, b1, W2, b2):
    D_h = W1.shape[1]
    D_out = W2.shape[1]

    src3 = edge_index[0].reshape(NW, NCH, CH)
    dst3 = edge_index[1].reshape(NW, NCH, CH)

    degp = _deg_kernel(dst3)
    d0 = degp[:NPAD].reshape(NPAD, 1)
    d1 = degp[NPAD:].reshape(NPAD, 1)

    y, dinv = pl.pallas_call(
        _tc1_body,
        out_shape=[
            jax.ShapeDtypeStruct((N, D_h), jnp.float32),
            jax.ShapeDtypeStruct((N, 1), jnp.float32),
        ],
    )(x, W1, d0, d1)

    r1 = _agg(y, src3, dst3)

    R = 2000
    _rows1 = pl.BlockSpec((R, 1), lambda i: (i, 0))

    y2 = pl.pallas_call(
        _tc2_body,
        grid=(N // R,),
        in_specs=[pl.BlockSpec((1, R, D_h), lambda i: (0, i, 0)),
                  pl.BlockSpec((1, R, D_h), lambda i: (1, i, 0)),
                  pl.BlockSpec((R, D_h), lambda i: (i, 0)),
                  _rows1,
                  pl.BlockSpec((1, D_h), lambda i: (0, 0)),
                  pl.BlockSpec((D_h, D_out), lambda i: (0, 0))],
        out_specs=pl.BlockSpec((R, D_h), lambda i: (i, 0)),
        out_shape=jax.ShapeDtypeStruct((N, D_h), jnp.float32),
    )(r1, r1, y, dinv, b1.reshape(1, D_h), W2)

    r2 = _agg(y2, src3, dst3)

    out = pl.pallas_call(
        _tc3_body,
        grid=(N // R,),
        in_specs=[pl.BlockSpec((1, R, D_h), lambda i: (0, i, 0)),
                  pl.BlockSpec((1, R, D_h), lambda i: (1, i, 0)),
                  pl.BlockSpec((R, D_h), lambda i: (i, 0)),
                  _rows1,
                  pl.BlockSpec((1, D_out), lambda i: (0, 0))],
        out_specs=pl.BlockSpec((R, D_out), lambda i: (i, 0)),
        out_shape=jax.ShapeDtypeStruct((N, D_out), jnp.float32),
    )(r2, r2, y2, dinv, b2.reshape(1, D_out))

    return out


# speedup vs baseline: 1.1520x; 1.0730x over previous
import functools

import jax
import jax.numpy as jnp
from jax import lax
from jax.experimental import pallas as pl
from jax.experimental.pallas import tpu as pltpu
from jax.experimental.pallas import tpu_sc as plsc

N = 10000
E = 320000
NC = 2
NS = 16
NW = NC * NS
EP = E // NW
CH = 80
NCH = 125
NPAD = 10240
ROWS_PER_TILE = NPAD // NS

_mesh = plsc.VectorSubcoreMesh(core_axis_name="c", subcore_axis_name="s")
_sc_params = pltpu.CompilerParams(use_tc_tiling_on_sc=False)


@functools.partial(
    pl.kernel,
    out_type=jax.ShapeDtypeStruct((NC * NPAD,), jnp.float32),
    mesh=_mesh,
    compiler_params=_sc_params,
    scratch_types=[
        pltpu.VMEM((NCH, CH), jnp.int32),
        pltpu.VMEM((CH,), jnp.float32),
        pltpu.VMEM((ROWS_PER_TILE,), jnp.float32),
        pltpu.VMEM_SHARED((NPAD,), jnp.float32),
    ],
)
def _deg_kernel(dst_hbm, out_hbm, didx, ones_v, zbuf, deg_sp):
    c = lax.axis_index("c")
    s = lax.axis_index("s")
    wid = c * NS + s

    def zstore(i, _):
        zbuf[pl.ds(i * 16, 16)] = jnp.zeros((16,), jnp.float32)
        return 0

    lax.fori_loop(0, ROWS_PER_TILE // 16, zstore, 0)

    def fill_ones(i, _):
        ones_v[pl.ds(i * 16, 16)] = jnp.ones((16,), jnp.float32)
        return 0

    lax.fori_loop(0, CH // 16, fill_ones, 0)

    pltpu.sync_copy(zbuf, deg_sp.at[pl.ds(s * ROWS_PER_TILE, ROWS_PER_TILE)])
    plsc.subcore_barrier()

    pltpu.sync_copy(dst_hbm.at[wid], didx)

    def body(j, _):
        pltpu.sync_copy(ones_v, deg_sp.at[didx.at[j]], add=True)
        return 0

    lax.fori_loop(0, NCH, body, 0)

    plsc.subcore_barrier()

    pltpu.sync_copy(deg_sp.at[pl.ds(s * ROWS_PER_TILE, ROWS_PER_TILE)], zbuf)
    pltpu.sync_copy(
        zbuf, out_hbm.at[pl.ds(c * NPAD + s * ROWS_PER_TILE, ROWS_PER_TILE)])


def _make_agg(W):
    zr = 32
    cols = W // 16

    @functools.partial(
        pl.kernel,
        out_type=jax.ShapeDtypeStruct((NC, NPAD, W), jnp.float32),
        mesh=_mesh,
        compiler_params=_sc_params,
        scratch_types=[
            pltpu.VMEM((NCH, CH), jnp.int32),
            pltpu.VMEM((NCH, CH), jnp.int32),
            pltpu.VMEM((CH, W), jnp.float32),
            pltpu.VMEM((CH, W), jnp.float32),
            pltpu.VMEM((zr, W), jnp.float32),
            pltpu.VMEM_SHARED((NPAD, W), jnp.float32),
            pltpu.SemaphoreType.DMA,
            pltpu.SemaphoreType.DMA,
        ],
    )
    def agg(y_hbm, src_hbm, dst_hbm, out_hbm,
            sidx, didx, rows0, rows1, zbuf, acc, sem0, sem1):
        c = lax.axis_index("c")
        s = lax.axis_index("s")
        wid = c * NS + s

        def zstore(t, _):
            zbuf[t // cols, pl.ds((t % cols) * 16, 16)] = jnp.zeros(
                (16,), jnp.float32)
            return 0

        lax.fori_loop(0, zr * cols, zstore, 0)

        pltpu.sync_copy(src_hbm.at[wid], sidx)
        pltpu.sync_copy(dst_hbm.at[wid], didx)

        for k in range(ROWS_PER_TILE // zr):
            pltpu.sync_copy(
                zbuf, acc.at[pl.ds(s * ROWS_PER_TILE + k * zr, zr)])
        plsc.subcore_barrier()

        def gather_start(j, buf, sem):
            return pltpu.async_copy(y_hbm.at[sidx.at[j]], buf, sem)

        def gather_wait(j, buf, sem):
            pltpu.make_async_copy(y_hbm.at[sidx.at[j]], buf, sem).wait()

        def scat(j, buf):
            pltpu.sync_copy(buf, acc.at[didx.at[j]], add=True)

        gather_start(0, rows0, sem0)

        def body(k, _):
            j0 = 2 * k
            gather_start(j0 + 1, rows1, sem1)
            gather_wait(j0, rows0, sem0)
            scat(j0, rows0)
            gather_start(j0 + 2, rows0, sem0)
            gather_wait(j0 + 1, rows1, sem1)
            scat(j0 + 1, rows1)
            return 0

        lax.fori_loop(0, (NCH - 1) // 2, body, 0)

        gather_wait(NCH - 1, rows0, sem0)
        scat(NCH - 1, rows0)

        plsc.subcore_barrier()

        for k in range(ROWS_PER_TILE // zr):
            base = s * ROWS_PER_TILE + k * zr
            pltpu.sync_copy(acc.at[pl.ds(base, zr)], zbuf)
            pltpu.sync_copy(zbuf, out_hbm.at[c, pl.ds(base, zr)])

    return agg


_agg_l1 = _make_agg(128)
_agg_l2 = _make_agg(64)


def _tc1_body(x_ref, w_ref, d0_ref, d1_ref, y_ref, dinv_ref):
    deg = d0_ref[0:N, :] + d1_ref[0:N, :] + 1.0
    dinv = lax.rsqrt(deg)
    dinv_ref[...] = dinv
    xw = jnp.dot(x_ref[...], w_ref[...],
                 preferred_element_type=jnp.float32,
                 precision=lax.Precision.HIGHEST)
    y_ref[...] = xw * dinv


_SELU_SCALE = 1.0507009873554804934193349852946
_SELU_ALPHA = 1.6732632423543772848170429916717


def _tc2_body(q0_ref, q1_ref, y_ref, dinv_ref, b1_ref, w2_ref, y2_ref):
    dinv = dinv_ref[...]
    agg = q0_ref[0] + q1_ref[0] + y_ref[...]
    pre = dinv * agg + b1_ref[...]
    h = _SELU_SCALE * jnp.where(
        pre > 0.0, pre, _SELU_ALPHA * (jnp.exp(pre) - 1.0))
    hw = jnp.dot(h, w2_ref[...],
                 preferred_element_type=jnp.float32,
                 precision=lax.Precision.HIGHEST)
    y2_ref[...] = hw * dinv


def _tc3_body(q0_ref, q1_ref, y2_ref, dinv_ref, b2_ref, out_ref):
    o = (dinv_ref[...] * (q0_ref[0] + q1_ref[0] + y2_ref[...])
         + b2_ref[...])
    m = jnp.max(o, axis=1, keepdims=True)
    lse = m + jnp.log(jnp.sum(jnp.exp(o - m), axis=1, keepdims=True))
    out_ref[...] = o - lse


def kernel(x, edge_index, W1, b1, W2, b2):
    D_h = W1.shape[1]
    D_out = W2.shape[1]

    src3 = edge_index[0].reshape(NW, NCH, CH)
    dst3 = edge_index[1].reshape(NW, NCH, CH)

    degp = _deg_kernel(dst3)
    d0 = degp[:NPAD].reshape(NPAD, 1)
    d1 = degp[NPAD:].reshape(NPAD, 1)

    y, dinv = pl.pallas_call(
        _tc1_body,
        out_shape=[
            jax.ShapeDtypeStruct((N, D_h), jnp.float32),
            jax.ShapeDtypeStruct((N, 1), jnp.float32),
        ],
    )(x, W1, d0, d1)

    r1 = _agg_l1(y, src3, dst3)

    R = 2000
    _rows1 = pl.BlockSpec((R, 1), lambda i: (i, 0))

    y2 = pl.pallas_call(
        _tc2_body,
        grid=(N // R,),
        in_specs=[pl.BlockSpec((1, R, D_h), lambda i: (0, i, 0)),
                  pl.BlockSpec((1, R, D_h), lambda i: (1, i, 0)),
                  pl.BlockSpec((R, D_h), lambda i: (i, 0)),
                  _rows1,
                  pl.BlockSpec((1, D_h), lambda i: (0, 0)),
                  pl.BlockSpec((D_h, D_out), lambda i: (0, 0))],
        out_specs=pl.BlockSpec((R, D_out), lambda i: (i, 0)),
        out_shape=jax.ShapeDtypeStruct((N, D_out), jnp.float32),
    )(r1, r1, y, dinv, b1.reshape(1, D_h), W2)

    r2 = _agg_l2(y2, src3, dst3)

    out = pl.pallas_call(
        _tc3_body,
        grid=(N // R,),
        in_specs=[pl.BlockSpec((1, R, D_out), lambda i: (0, i, 0)),
                  pl.BlockSpec((1, R, D_out), lambda i: (1, i, 0)),
                  pl.BlockSpec((R, D_out), lambda i: (i, 0)),
                  _rows1,
                  pl.BlockSpec((1, D_out), lambda i: (0, 0))],
        out_specs=pl.BlockSpec((R, D_out), lambda i: (i, 0)),
        out_shape=jax.ShapeDtypeStruct((N, D_out), jnp.float32),
    )(r2, r2, y2, dinv, b2.reshape(1, D_out))

    return out
